# baseline (speedup 1.0000x reference)
import functools

import jax
import jax.numpy as jnp
import numpy as np
from jax import lax
from jax.experimental import pallas as pl
from jax.experimental.pallas import tpu as pltpu
from jax.experimental.pallas import tpu_sc as plsc

_N = 10000
_NPAD = 10240
_E = 320000
_DIN = 128
_H = 64
_G = 64
_NC = 2
_NS = 16
_NW = _NC * _NS
_EPW = _E // _NW
_K = 6400
_NCHUNK = _E // _K
_PACK = 16384
_UNROLL = 16
_BLK = 512
_NBLK = _NPAD // _BLK
_BN_R = float(1.0 / np.sqrt(1.0 + 1e-5))

_SC_PARAMS = pltpu.CompilerParams(
    use_tc_tiling_on_sc=False, needs_layout_passes=False
)


def _sc_mesh():
    return plsc.VectorSubcoreMesh(core_axis_name="c", subcore_axis_name="s")


def _deg_body(pk_hbm, out_hbm, idxv, histv):
    c = lax.axis_index("c")
    sid = lax.axis_index("s")
    w = c * _NS + sid

    def _z(i, carry):
        histv[pl.ds(i * 16, 16)] = jnp.zeros((16,), jnp.float32)
        return carry

    lax.fori_loop(0, _NPAD // 16, _z, 0)
    pltpu.sync_copy(pk_hbm.at[pl.ds(w * _EPW, _EPW)], idxv)

    ones16 = jnp.ones((16,), jnp.float32)

    def _g(i, carry):
        dvs = []
        for u in range(8):
            ev = idxv[pl.ds((i * 8 + u) * 16, 16)]
            dvs.append(lax.shift_right_logical(ev, 14))
        for dv in dvs:
            plsc.addupdate_scatter(histv, [dv], ones16)
        return carry

    lax.fori_loop(0, _EPW // 128, _g, 0)
    pltpu.sync_copy(histv, out_hbm.at[w])


_deg_call = pl.kernel(
    _deg_body,
    out_type=jax.ShapeDtypeStruct((_NW, _NPAD), jnp.float32),
    mesh=_sc_mesh(),
    scratch_types=[
        pltpu.VMEM((_EPW,), jnp.int32),
        pltpu.VMEM((_NPAD,), jnp.float32),
    ],
    compiler_params=_SC_PARAMS,
)


def _edge_body(pk_hbm, s_hbm, out_hbm, sbuf, abuf, ib0, ib1, sem0, sem1):
    c = lax.axis_index("c")
    sid = lax.axis_index("s")
    w = c * _NS + sid

    pltpu.sync_copy(s_hbm.at[pl.ds(2 * w, 2)], sbuf)

    def _z(i, carry):
        for f in range(2):
            abuf[f, pl.ds(i * 16, 16)] = jnp.zeros((16,), jnp.float32)
        return carry

    lax.fori_loop(0, _NPAD // 16, _z, 0)

    ibs = (ib0, ib1)
    sems = (sem0, sem1)
    pltpu.async_copy(pk_hbm.at[pl.ds(0, _K)], ib0, sem0)

    z16 = jnp.zeros((16,), jnp.int32)
    o16 = jnp.ones((16,), jnp.int32)
    m16 = jnp.full((16,), _PACK - 1, jnp.int32)

    def _round(r, carry):
        for b in range(2):
            cidx = r * 2 + b
            pltpu.make_async_copy(
                pk_hbm.at[pl.ds(0, _K)], ibs[b], sems[b]
            ).wait()
            nxt = jnp.minimum((cidx + 1) * _K, _E - _K)

            @pl.when(cidx + 1 < _NCHUNK)
            def _():
                pltpu.async_copy(
                    pk_hbm.at[pl.ds(nxt, _K)], ibs[b ^ 1], sems[b ^ 1]
                )

            ib = ibs[b]

            def _g(i, carry2):
                dvs, vals = [], []
                for u in range(_UNROLL):
                    ev = ib[pl.ds((i * _UNROLL + u) * 16, 16)]
                    sv = jnp.bitwise_and(ev, m16)
                    dvs.append(lax.shift_right_logical(ev, 14))
                    vals.append((plsc.load_gather(sbuf, [z16, sv]),
                                 plsc.load_gather(sbuf, [o16, sv])))
                for dv, (v0, v1) in zip(dvs, vals):
                    plsc.addupdate_scatter(abuf, [z16, dv], v0)
                    plsc.addupdate_scatter(abuf, [o16, dv], v1)
                return carry2

            lax.fori_loop(0, _K // (16 * _UNROLL), _g, 0)
        return carry

    lax.fori_loop(0, _NCHUNK // 2, _round, 0)
    pltpu.sync_copy(abuf, out_hbm.at[pl.ds(2 * w, 2)])


_edge_call = pl.kernel(
    _edge_body,
    out_type=jax.ShapeDtypeStruct((_H, _NPAD), jnp.float32),
    mesh=_sc_mesh(),
    scratch_types=[
        pltpu.VMEM((2, _NPAD), jnp.float32),
        pltpu.VMEM((2, _NPAD), jnp.float32),
        pltpu.VMEM((_K,), jnp.int32),
        pltpu.VMEM((_K,), jnp.int32),
        pltpu.SemaphoreType.DMA,
        pltpu.SemaphoreType.DMA,
    ],
    compiler_params=_SC_PARAMS,
)


def _prologue_body(x_ref, h_ref, w_ref, o_ref, d_ref):
    deg = 1.0 + jnp.sum(h_ref[...], axis=0, keepdims=True)
    dinv = lax.rsqrt(deg)
    d_ref[...] = dinv
    dn = (((0,), (0,)), ((), ()))
    o_ref[...] = (
        lax.dot_general(w_ref[...], x_ref[...], dn,
                        preferred_element_type=jnp.float32) * dinv
    )


_prologue_call = pl.pallas_call(
    _prologue_body,
    grid=(_NBLK,),
    in_specs=[
        pl.BlockSpec((_DIN, _BLK), lambda i: (0, i)),
        pl.BlockSpec((_NW, _BLK), lambda i: (0, i)),
        pl.BlockSpec((_DIN, _H), lambda i: (0, 0)),
    ],
    out_specs=[
        pl.BlockSpec((_H, _BLK), lambda i: (0, i)),
        pl.BlockSpec((1, _BLK), lambda i: (0, i)),
    ],
    out_shape=[
        jax.ShapeDtypeStruct((_H, _NPAD), jnp.float32),
        jax.ShapeDtypeStruct((1, _NPAD), jnp.float32),
    ],
)


def _mid_body(a, s, d_ref, ga, cb, w_ref, o_ref):
    dinv = d_ref[...]
    h = jnp.maximum(dinv * (a[...] + s[...]) * ga[...] + cb[...], 0.0)
    dn = (((0,), (0,)), ((), ()))
    o_ref[...] = (
        lax.dot_general(w_ref[...], h, dn,
                        preferred_element_type=jnp.float32) * dinv
    )


_mid_call = pl.pallas_call(
    _mid_body,
    grid=(_NBLK,),
    in_specs=[
        pl.BlockSpec((_H, _BLK), lambda i: (0, i)),
        pl.BlockSpec((_H, _BLK), lambda i: (0, i)),
        pl.BlockSpec((1, _BLK), lambda i: (0, i)),
        pl.BlockSpec((_H, 1), lambda i: (0, 0)),
        pl.BlockSpec((_H, 1), lambda i: (0, 0)),
        pl.BlockSpec((_H, _H), lambda i: (0, 0)),
    ],
    out_specs=pl.BlockSpec((_H, _BLK), lambda i: (0, i)),
    out_shape=jax.ShapeDtypeStruct((_H, _NPAD), jnp.float32),
)


def _pool_body(a, s, d_ref, ga, cb, b_ref,
               m1w, m1b, m2w, m2b, m3w, m3b, ow, ob,
               t1w, t1b, t2w, t2b, t3w, t3b, xl, bias,
               o_ref, sums, cnts):
    i = pl.program_id(0)
    f32 = jnp.float32
    dinv = d_ref[...]
    h3 = jnp.maximum(dinv * (a[...] + s[...]) * ga[...] + cb[...], 0.0)
    gid = lax.broadcasted_iota(jnp.int32, (_G, _BLK), 0)
    oh = (b_ref[...] == gid).astype(f32)
    dn1 = (((1,), (1,)), ((), ()))
    ps = lax.dot_general(h3, oh, dn1, preferred_element_type=f32)
    pc = lax.dot_general(
        jnp.ones((_H, _BLK), f32), oh, dn1, preferred_element_type=f32
    )

    @pl.when(i == 0)
    def _():
        sums[...] = jnp.zeros_like(sums)
        cnts[...] = jnp.zeros_like(cnts)

    sums[...] += ps
    cnts[...] += pc

    @pl.when(i == _NBLK - 1)
    def _():
        dn = (((0,), (0,)), ((), ()))
        pooled = sums[...] / jnp.maximum(cnts[...], 1.0)
        z = jnp.maximum(
            lax.dot_general(m1w[...], pooled, dn, preferred_element_type=f32)
            + m1b[...], 0.0)
        z = jnp.maximum(
            lax.dot_general(m2w[...], z, dn, preferred_element_type=f32)
            + m2b[...], 0.0)
        z = jnp.maximum(
            lax.dot_general(m3w[...], z, dn, preferred_element_type=f32)
            + m3b[...], 0.0)
        bf = (lax.dot_general(z, ow[...], dn, preferred_element_type=f32)
              + ob[...])
        t = jnp.maximum(
            jnp.dot(xl[...], t1w[...], preferred_element_type=f32)
            + t1b[...], 0.0)
        t = jnp.maximum(
            jnp.dot(t, t2w[...], preferred_element_type=f32) + t2b[...], 0.0)
        tf = jnp.dot(t, t3w[...], preferred_element_type=f32) + t3b[...]
        o_ref[...] = bf * tf + bias[...]


def _const_spec(shape):
    return pl.BlockSpec(shape, lambda i: tuple(0 for _ in shape))


_pool_call = pl.pallas_call(
    _pool_body,
    grid=(_NBLK,),
    in_specs=[
        pl.BlockSpec((_H, _BLK), lambda i: (0, i)),
        pl.BlockSpec((_H, _BLK), lambda i: (0, i)),
        pl.BlockSpec((1, _BLK), lambda i: (0, i)),
        _const_spec((_H, 1)),
        _const_spec((_H, 1)),
        pl.BlockSpec((1, _BLK), lambda i: (0, i)),
        _const_spec((_H, _H)),
        _const_spec((_H, 1)),
        _const_spec((_H, 32)),
        _const_spec((32, 1)),
        _const_spec((32, 16)),
        _const_spec((16, 1)),
        _const_spec((16, 2)),
        _const_spec((1, 2)),
        _const_spec((2, 128)),
        _const_spec((1, 128)),
        _const_spec((128, 256)),
        _const_spec((1, 256)),
        _const_spec((256, 2)),
        _const_spec((1, 2)),
        _const_spec((_G, 2)),
        _const_spec((1, 2)),
    ],
    out_specs=pl.BlockSpec((_G, 2), lambda i: (0, 0)),
    out_shape=jax.ShapeDtypeStruct((_G, 2), jnp.float32),
    scratch_shapes=[
        pltpu.VMEM((_H, _G), jnp.float32),
        pltpu.VMEM((_H, _G), jnp.float32),
    ],
)


def kernel(x, edge_index, batch, x_loc, params):
    p = params
    pk = edge_index[0] + edge_index[1] * _PACK
    xT = jnp.pad(x, ((0, _NPAD - _N), (0, 0))).T
    bp = jnp.pad(batch, (0, _NPAD - _N), constant_values=_G).reshape(1, _NPAD)

    def fold(g, be, b):
        ga = (g * _BN_R).reshape(_H, 1)
        cb = (b * g * _BN_R + be).reshape(_H, 1)
        return ga, cb

    ga1, cb1 = fold(p["g1"], p["be1"], p["b1"])
    ga2, cb2 = fold(p["g2"], p["be2"], p["b2"])
    ga3, cb3 = fold(p["g3"], p["be3"], p["b3"])

    hist = _deg_call(pk)
    s1, dinv = _prologue_call(xT, hist, p["W1"])
    a1 = _edge_call(pk, s1)
    s2 = _mid_call(a1, s1, dinv, ga1, cb1, p["W2"])
    a2 = _edge_call(pk, s2)
    s3 = _mid_call(a2, s2, dinv, ga2, cb2, p["W3"])
    a3 = _edge_call(pk, s3)
    out = _pool_call(
        a3, s3, dinv, ga3, cb3, bp,
        p["m1W"], p["m1b"].reshape(-1, 1),
        p["m2W"], p["m2b"].reshape(-1, 1),
        p["m3W"], p["m3b"].reshape(-1, 1),
        p["oW"], p["ob"].reshape(1, -1),
        p["t1W"], p["t1b"].reshape(1, -1),
        p["t2W"], p["t2b"].reshape(1, -1),
        p["t3W"], p["t3b"].reshape(1, -1),
        x_loc, p["bias"].reshape(1, -1),
    )
    return out


# speedup vs baseline: 1.1350x; 1.1350x over previous
import functools

import jax
import jax.numpy as jnp
import numpy as np
from jax import lax
from jax.experimental import pallas as pl
from jax.experimental.pallas import tpu as pltpu
from jax.experimental.pallas import tpu_sc as plsc

_N = 10000
_NPAD = 10240
_E = 320000
_DIN = 128
_H = 64
_G = 64
_NC = 2
_NS = 16
_NW = _NC * _NS
_EPW = _E // _NW
_K = 6400
_NCHUNK = _E // _K
_PACK = 16384
_UNROLL = 16
_BLK = 512
_NBLK = _NPAD // _BLK
_BN_R = float(1.0 / np.sqrt(1.0 + 1e-5))

_SC_PARAMS = pltpu.CompilerParams(
    use_tc_tiling_on_sc=False, needs_layout_passes=False
)


def _sc_mesh():
    return plsc.VectorSubcoreMesh(core_axis_name="c", subcore_axis_name="s")


def _deg_body(pk_hbm, out_hbm, idxv, histv):
    c = lax.axis_index("c")
    sid = lax.axis_index("s")
    w = c * _NS + sid

    def _z(i, carry):
        histv[pl.ds(i * 16, 16)] = jnp.zeros((16,), jnp.float32)
        return carry

    lax.fori_loop(0, _NPAD // 16, _z, 0)
    pltpu.sync_copy(pk_hbm.at[pl.ds(w * _EPW, _EPW)], idxv)

    ones16 = jnp.ones((16,), jnp.float32)

    def _g(i, carry):
        dvs = []
        for u in range(8):
            ev = idxv[pl.ds((i * 8 + u) * 16, 16)]
            dvs.append(lax.shift_right_logical(ev, 14))
        for dv in dvs:
            plsc.addupdate_scatter(histv, [dv], ones16)
        return carry

    lax.fori_loop(0, _EPW // 128, _g, 0)
    pltpu.sync_copy(histv, out_hbm.at[w])


_deg_call = pl.kernel(
    _deg_body,
    out_type=jax.ShapeDtypeStruct((_NW, _NPAD), jnp.float32),
    mesh=_sc_mesh(),
    scratch_types=[
        pltpu.VMEM((_EPW,), jnp.int32),
        pltpu.VMEM((_NPAD,), jnp.float32),
    ],
    compiler_params=_SC_PARAMS,
)


def _edge_body(pk_hbm, s_hbm, out_hbm, sbuf, abuf, ib0, ib1, sem0, sem1):
    c = lax.axis_index("c")
    sid = lax.axis_index("s")
    w = c * _NS + sid

    pltpu.sync_copy(s_hbm.at[w], sbuf)

    def _z(i, carry):
        for f in range(2):
            abuf[f, pl.ds(i * 16, 16)] = jnp.zeros((16,), jnp.float32)
        return carry

    lax.fori_loop(0, _NPAD // 16, _z, 0)

    ibs = (ib0, ib1)
    sems = (sem0, sem1)
    pltpu.async_copy(pk_hbm.at[pl.ds(0, _K)], ib0, sem0)

    z16 = jnp.zeros((16,), jnp.int32)
    o16 = jnp.ones((16,), jnp.int32)
    m16 = jnp.full((16,), _PACK - 1, jnp.int32)
    mhi = jnp.full((16,), -65536, jnp.int32)

    def _round(r, carry):
        for b in range(2):
            cidx = r * 2 + b
            pltpu.make_async_copy(
                pk_hbm.at[pl.ds(0, _K)], ibs[b], sems[b]
            ).wait()
            nxt = jnp.minimum((cidx + 1) * _K, _E - _K)

            @pl.when(cidx + 1 < _NCHUNK)
            def _():
                pltpu.async_copy(
                    pk_hbm.at[pl.ds(nxt, _K)], ibs[b ^ 1], sems[b ^ 1]
                )

            ib = ibs[b]

            def _g(i, carry2):
                dvs, vals = [], []
                for u in range(_UNROLL):
                    ev = ib[pl.ds((i * _UNROLL + u) * 16, 16)]
                    sv = jnp.bitwise_and(ev, m16)
                    dvs.append(lax.shift_right_logical(ev, 14))
                    gv = plsc.load_gather(sbuf, [sv])
                    f0 = plsc.bitcast(lax.shift_left(gv, 16), jnp.float32)
                    f1 = plsc.bitcast(jnp.bitwise_and(gv, mhi), jnp.float32)
                    vals.append((f0, f1))
                for dv, (v0, v1) in zip(dvs, vals):
                    plsc.addupdate_scatter(abuf, [z16, dv], v0)
                    plsc.addupdate_scatter(abuf, [o16, dv], v1)
                return carry2

            lax.fori_loop(0, _K // (16 * _UNROLL), _g, 0)
        return carry

    lax.fori_loop(0, _NCHUNK // 2, _round, 0)
    pltpu.sync_copy(abuf.at[0], out_hbm.at[w])
    pltpu.sync_copy(abuf.at[1], out_hbm.at[w + _NW])


_edge_call = pl.kernel(
    _edge_body,
    out_type=jax.ShapeDtypeStruct((_H, _NPAD), jnp.float32),
    mesh=_sc_mesh(),
    scratch_types=[
        pltpu.VMEM((_NPAD,), jnp.int32),
        pltpu.VMEM((2, _NPAD), jnp.float32),
        pltpu.VMEM((_K,), jnp.int32),
        pltpu.VMEM((_K,), jnp.int32),
        pltpu.SemaphoreType.DMA,
        pltpu.SemaphoreType.DMA,
    ],
    compiler_params=_SC_PARAMS,
)


def _pack_bf16(s):
    sb = lax.bitcast_convert_type(s, jnp.int32)
    rb = lax.shift_right_logical(
        sb + 0x7FFF + jnp.bitwise_and(lax.shift_right_logical(sb, 16), 1), 16
    )
    return jnp.bitwise_or(rb[: _H // 2, :],
                          lax.shift_left(rb[_H // 2:, :], 16))


def _unpack_bf16(pk):
    lo = lax.bitcast_convert_type(lax.shift_left(pk, 16), jnp.float32)
    hi = lax.bitcast_convert_type(
        jnp.bitwise_and(pk, jnp.int32(-65536)), jnp.float32
    )
    return jnp.concatenate([lo, hi], axis=0)


def _prologue_body(x_ref, h_ref, w_ref, o_ref, d_ref):
    deg = 1.0 + jnp.sum(h_ref[...], axis=0, keepdims=True)
    dinv = lax.rsqrt(deg)
    d_ref[...] = dinv
    dn = (((0,), (0,)), ((), ()))
    s = lax.dot_general(w_ref[...], x_ref[...], dn,
                        preferred_element_type=jnp.float32) * dinv
    o_ref[...] = _pack_bf16(s)


_prologue_call = pl.pallas_call(
    _prologue_body,
    grid=(_NBLK,),
    in_specs=[
        pl.BlockSpec((_DIN, _BLK), lambda i: (0, i)),
        pl.BlockSpec((_NW, _BLK), lambda i: (0, i)),
        pl.BlockSpec((_DIN, _H), lambda i: (0, 0)),
    ],
    out_specs=[
        pl.BlockSpec((_H // 2, _BLK), lambda i: (0, i)),
        pl.BlockSpec((1, _BLK), lambda i: (0, i)),
    ],
    out_shape=[
        jax.ShapeDtypeStruct((_H // 2, _NPAD), jnp.int32),
        jax.ShapeDtypeStruct((1, _NPAD), jnp.float32),
    ],
)


def _mid_body(a, s, d_ref, ga, cb, w_ref, o_ref):
    dinv = d_ref[...]
    sf = _unpack_bf16(s[...])
    h = jnp.maximum(dinv * (a[...] + sf) * ga[...] + cb[...], 0.0)
    dn = (((0,), (0,)), ((), ()))
    o_ref[...] = _pack_bf16(
        lax.dot_general(w_ref[...], h, dn,
                        preferred_element_type=jnp.float32) * dinv
    )


_mid_call = pl.pallas_call(
    _mid_body,
    grid=(_NBLK,),
    in_specs=[
        pl.BlockSpec((_H, _BLK), lambda i: (0, i)),
        pl.BlockSpec((_H // 2, _BLK), lambda i: (0, i)),
        pl.BlockSpec((1, _BLK), lambda i: (0, i)),
        pl.BlockSpec((_H, 1), lambda i: (0, 0)),
        pl.BlockSpec((_H, 1), lambda i: (0, 0)),
        pl.BlockSpec((_H, _H), lambda i: (0, 0)),
    ],
    out_specs=pl.BlockSpec((_H // 2, _BLK), lambda i: (0, i)),
    out_shape=jax.ShapeDtypeStruct((_H // 2, _NPAD), jnp.int32),
)


def _pool_body(a, s, d_ref, ga, cb, b_ref,
               m1w, m1b, m2w, m2b, m3w, m3b, ow, ob,
               t1w, t1b, t2w, t2b, t3w, t3b, xl, bias,
               o_ref, sums, cnts):
    i = pl.program_id(0)
    f32 = jnp.float32
    dinv = d_ref[...]
    sf = _unpack_bf16(s[...])
    h3 = jnp.maximum(dinv * (a[...] + sf) * ga[...] + cb[...], 0.0)
    gid = lax.broadcasted_iota(jnp.int32, (_G, _BLK), 0)
    oh = (b_ref[...] == gid).astype(f32)
    dn1 = (((1,), (1,)), ((), ()))
    ps = lax.dot_general(h3, oh, dn1, preferred_element_type=f32)
    pc = lax.dot_general(
        jnp.ones((_H, _BLK), f32), oh, dn1, preferred_element_type=f32
    )

    @pl.when(i == 0)
    def _():
        sums[...] = jnp.zeros_like(sums)
        cnts[...] = jnp.zeros_like(cnts)

    sums[...] += ps
    cnts[...] += pc

    @pl.when(i == _NBLK - 1)
    def _():
        dn = (((0,), (0,)), ((), ()))
        pooled = sums[...] / jnp.maximum(cnts[...], 1.0)
        z = jnp.maximum(
            lax.dot_general(m1w[...], pooled, dn, preferred_element_type=f32)
            + m1b[...], 0.0)
        z = jnp.maximum(
            lax.dot_general(m2w[...], z, dn, preferred_element_type=f32)
            + m2b[...], 0.0)
        z = jnp.maximum(
            lax.dot_general(m3w[...], z, dn, preferred_element_type=f32)
            + m3b[...], 0.0)
        bf = (lax.dot_general(z, ow[...], dn, preferred_element_type=f32)
              + ob[...])
        t = jnp.maximum(
            jnp.dot(xl[...], t1w[...], preferred_element_type=f32)
            + t1b[...], 0.0)
        t = jnp.maximum(
            jnp.dot(t, t2w[...], preferred_element_type=f32) + t2b[...], 0.0)
        tf = jnp.dot(t, t3w[...], preferred_element_type=f32) + t3b[...]
        o_ref[...] = bf * tf + bias[...]


def _const_spec(shape):
    return pl.BlockSpec(shape, lambda i: tuple(0 for _ in shape))


_pool_call = pl.pallas_call(
    _pool_body,
    grid=(_NBLK,),
    in_specs=[
        pl.BlockSpec((_H, _BLK), lambda i: (0, i)),
        pl.BlockSpec((_H // 2, _BLK), lambda i: (0, i)),
        pl.BlockSpec((1, _BLK), lambda i: (0, i)),
        _const_spec((_H, 1)),
        _const_spec((_H, 1)),
        pl.BlockSpec((1, _BLK), lambda i: (0, i)),
        _const_spec((_H, _H)),
        _const_spec((_H, 1)),
        _const_spec((_H, 32)),
        _const_spec((32, 1)),
        _const_spec((32, 16)),
        _const_spec((16, 1)),
        _const_spec((16, 2)),
        _const_spec((1, 2)),
        _const_spec((2, 128)),
        _const_spec((1, 128)),
        _const_spec((128, 256)),
        _const_spec((1, 256)),
        _const_spec((256, 2)),
        _const_spec((1, 2)),
        _const_spec((_G, 2)),
        _const_spec((1, 2)),
    ],
    out_specs=pl.BlockSpec((_G, 2), lambda i: (0, 0)),
    out_shape=jax.ShapeDtypeStruct((_G, 2), jnp.float32),
    scratch_shapes=[
        pltpu.VMEM((_H, _G), jnp.float32),
        pltpu.VMEM((_H, _G), jnp.float32),
    ],
)


def kernel(x, edge_index, batch, x_loc, params):
    p = params
    pk = edge_index[0] + edge_index[1] * _PACK
    xT = jnp.pad(x, ((0, _NPAD - _N), (0, 0))).T
    bp = jnp.pad(batch, (0, _NPAD - _N), constant_values=_G).reshape(1, _NPAD)

    def fold(g, be, b):
        ga = (g * _BN_R).reshape(_H, 1)
        cb = (b * g * _BN_R + be).reshape(_H, 1)
        return ga, cb

    ga1, cb1 = fold(p["g1"], p["be1"], p["b1"])
    ga2, cb2 = fold(p["g2"], p["be2"], p["b2"])
    ga3, cb3 = fold(p["g3"], p["be3"], p["b3"])

    hist = _deg_call(pk)
    s1, dinv = _prologue_call(xT, hist, p["W1"])
    a1 = _edge_call(pk, s1)
    s2 = _mid_call(a1, s1, dinv, ga1, cb1, p["W2"])
    a2 = _edge_call(pk, s2)
    s3 = _mid_call(a2, s2, dinv, ga2, cb2, p["W3"])
    a3 = _edge_call(pk, s3)
    out = _pool_call(
        a3, s3, dinv, ga3, cb3, bp,
        p["m1W"], p["m1b"].reshape(-1, 1),
        p["m2W"], p["m2b"].reshape(-1, 1),
        p["m3W"], p["m3b"].reshape(-1, 1),
        p["oW"], p["ob"].reshape(1, -1),
        p["t1W"], p["t1b"].reshape(1, -1),
        p["t2W"], p["t2b"].reshape(1, -1),
        p["t3W"], p["t3b"].reshape(1, -1),
        x_loc, p["bias"].reshape(1, -1),
    )
    return out


# speedup vs baseline: 1.1350x; 1.0000x over previous
import jax
import jax.numpy as jnp
import numpy as np
from jax import lax
from jax.experimental import pallas as pl
from jax.experimental.pallas import tpu as pltpu
from jax.experimental.pallas import tpu_sc as plsc

_N = 10000
_NPAD = 10240
_E = 320000
_DIN = 128
_H = 64
_G = 64
_NC = 2
_NS = 16
_NW = _NC * _NS
_EPW = _E // _NW
_K = 6400
_NCHUNK = _E // _K
_PACK = 16384
_UNROLL = 16
_BLK = 512
_NBLK = _NPAD // _BLK
_BN_R = float(1.0 / np.sqrt(1.0 + 1e-5))

_SC_PARAMS = pltpu.CompilerParams(
    use_tc_tiling_on_sc=False, needs_layout_passes=False
)


def _sc_mesh():
    return plsc.VectorSubcoreMesh(core_axis_name="c", subcore_axis_name="s")


def _deg_body(pk_hbm, out_hbm, idxv, histv):
    c = lax.axis_index("c")
    sid = lax.axis_index("s")
    w = c * _NS + sid

    def _z(i, carry):
        histv[pl.ds(i * 16, 16)] = jnp.zeros((16,), jnp.float32)
        return carry

    lax.fori_loop(0, _NPAD // 16, _z, 0)
    pltpu.sync_copy(pk_hbm.at[pl.ds(w * _EPW, _EPW)], idxv)

    ones16 = jnp.ones((16,), jnp.float32)

    def _g(i, carry):
        dvs = []
        for u in range(8):
            ev = idxv[pl.ds((i * 8 + u) * 16, 16)]
            dvs.append(lax.shift_right_logical(ev, 14))
        for dv in dvs:
            plsc.addupdate_scatter(histv, [dv], ones16)
        return carry

    lax.fori_loop(0, _EPW // 128, _g, 0)
    pltpu.sync_copy(histv, out_hbm.at[w])


_deg_call = pl.kernel(
    _deg_body,
    out_type=jax.ShapeDtypeStruct((_NW, _NPAD), jnp.float32),
    mesh=_sc_mesh(),
    scratch_types=[
        pltpu.VMEM((_EPW,), jnp.int32),
        pltpu.VMEM((_NPAD,), jnp.float32),
    ],
    compiler_params=_SC_PARAMS,
)


def _edge_body(pk_hbm, s_hbm, out_hbm, sbuf, abuf, ib0, ib1, sem0, sem1):
    c = lax.axis_index("c")
    sid = lax.axis_index("s")
    w = c * _NS + sid

    pltpu.sync_copy(s_hbm.at[w], sbuf)

    def _z(i, carry):
        for f in range(2):
            abuf[f, pl.ds(i * 16, 16)] = jnp.zeros((16,), jnp.float32)
        return carry

    lax.fori_loop(0, _NPAD // 16, _z, 0)

    ibs = (ib0, ib1)
    sems = (sem0, sem1)
    pltpu.async_copy(pk_hbm.at[pl.ds(0, _K)], ib0, sem0)

    z16 = jnp.zeros((16,), jnp.int32)
    o16 = jnp.ones((16,), jnp.int32)
    m16 = jnp.full((16,), _PACK - 1, jnp.int32)
    mhi = jnp.full((16,), -65536, jnp.int32)

    def _round(r, carry):
        for b in range(2):
            cidx = r * 2 + b
            pltpu.make_async_copy(
                pk_hbm.at[pl.ds(0, _K)], ibs[b], sems[b]
            ).wait()
            nxt = jnp.minimum((cidx + 1) * _K, _E - _K)

            @pl.when(cidx + 1 < _NCHUNK)
            def _():
                pltpu.async_copy(
                    pk_hbm.at[pl.ds(nxt, _K)], ibs[b ^ 1], sems[b ^ 1]
                )

            ib = ibs[b]

            def _g(i, carry2):
                dvs, vals = [], []
                for u in range(_UNROLL):
                    ev = ib[pl.ds((i * _UNROLL + u) * 16, 16)]
                    sv = jnp.bitwise_and(ev, m16)
                    dvs.append(lax.shift_right_logical(ev, 14))
                    gv = plsc.load_gather(sbuf, [sv])
                    f0 = plsc.bitcast(lax.shift_left(gv, 16), jnp.float32)
                    f1 = plsc.bitcast(jnp.bitwise_and(gv, mhi), jnp.float32)
                    vals.append((f0, f1))
                for dv, (v0, v1) in zip(dvs, vals):
                    plsc.addupdate_scatter(abuf, [z16, dv], v0)
                    plsc.addupdate_scatter(abuf, [o16, dv], v1)
                return carry2

            lax.fori_loop(0, _K // (16 * _UNROLL), _g, 0)
        return carry

    lax.fori_loop(0, _NCHUNK // 2, _round, 0)
    pltpu.sync_copy(abuf.at[0], out_hbm.at[w])
    pltpu.sync_copy(abuf.at[1], out_hbm.at[w + _NW])


_edge_call = pl.kernel(
    _edge_body,
    out_type=jax.ShapeDtypeStruct((_H, _NPAD), jnp.float32),
    mesh=_sc_mesh(),
    scratch_types=[
        pltpu.VMEM((_NPAD,), jnp.int32),
        pltpu.VMEM((2, _NPAD), jnp.float32),
        pltpu.VMEM((_K,), jnp.int32),
        pltpu.VMEM((_K,), jnp.int32),
        pltpu.SemaphoreType.DMA,
        pltpu.SemaphoreType.DMA,
    ],
    compiler_params=_SC_PARAMS,
)


def _pack_bf16(s):
    sb = lax.bitcast_convert_type(s, jnp.int32)
    rb = lax.shift_right_logical(
        sb + 0x7FFF + jnp.bitwise_and(lax.shift_right_logical(sb, 16), 1), 16
    )
    return jnp.bitwise_or(rb[: _H // 2, :],
                          lax.shift_left(rb[_H // 2:, :], 16))


def _unpack_bf16(pk):
    lo = lax.bitcast_convert_type(lax.shift_left(pk, 16), jnp.float32)
    hi = lax.bitcast_convert_type(
        jnp.bitwise_and(pk, jnp.int32(-65536)), jnp.float32
    )
    return jnp.concatenate([lo, hi], axis=0)


def _prologue_body(x_ref, h_ref, w_ref, o_ref, d_ref):
    deg = 1.0 + jnp.sum(h_ref[...], axis=0, keepdims=True)
    dinv = lax.rsqrt(deg)
    d_ref[...] = dinv
    dn = (((0,), (0,)), ((), ()))
    s = lax.dot_general(w_ref[...], x_ref[...], dn,
                        preferred_element_type=jnp.float32) * dinv
    o_ref[...] = _pack_bf16(s)


_prologue_call = pl.pallas_call(
    _prologue_body,
    grid=(_NBLK,),
    in_specs=[
        pl.BlockSpec((_DIN, _BLK), lambda i: (0, i)),
        pl.BlockSpec((_NW, _BLK), lambda i: (0, i)),
        pl.BlockSpec((_DIN, _H), lambda i: (0, 0)),
    ],
    out_specs=[
        pl.BlockSpec((_H // 2, _BLK), lambda i: (0, i)),
        pl.BlockSpec((1, _BLK), lambda i: (0, i)),
    ],
    out_shape=[
        jax.ShapeDtypeStruct((_H // 2, _NPAD), jnp.int32),
        jax.ShapeDtypeStruct((1, _NPAD), jnp.float32),
    ],
)


def _mid_body(a, s, d_ref, ga, cb, w_ref, o_ref):
    dinv = d_ref[...]
    sf = _unpack_bf16(s[...])
    h = jnp.maximum(dinv * (a[...] + sf) * ga[...] + cb[...], 0.0)
    dn = (((0,), (0,)), ((), ()))
    o_ref[...] = _pack_bf16(
        lax.dot_general(w_ref[...], h, dn,
                        preferred_element_type=jnp.float32) * dinv
    )


_mid_call = pl.pallas_call(
    _mid_body,
    grid=(_NBLK,),
    in_specs=[
        pl.BlockSpec((_H, _BLK), lambda i: (0, i)),
        pl.BlockSpec((_H // 2, _BLK), lambda i: (0, i)),
        pl.BlockSpec((1, _BLK), lambda i: (0, i)),
        pl.BlockSpec((_H, 1), lambda i: (0, 0)),
        pl.BlockSpec((_H, 1), lambda i: (0, 0)),
        pl.BlockSpec((_H, _H), lambda i: (0, 0)),
    ],
    out_specs=pl.BlockSpec((_H // 2, _BLK), lambda i: (0, i)),
    out_shape=jax.ShapeDtypeStruct((_H // 2, _NPAD), jnp.int32),
)


def _pool_body(a, s, d_ref, ga, cb, b_ref,
               m1w, m1b, m2w, m2b, m3w, m3b, ow, ob,
               t1w, t1b, t2w, t2b, t3w, t3b, xl, bias,
               o_ref, sums, cnts):
    i = pl.program_id(0)
    f32 = jnp.float32
    dinv = d_ref[...]
    sf = _unpack_bf16(s[...])
    h3 = jnp.maximum(dinv * (a[...] + sf) * ga[...] + cb[...], 0.0)
    gid = lax.broadcasted_iota(jnp.int32, (_G, _BLK), 0)
    oh = (b_ref[...] == gid).astype(f32)
    dn1 = (((1,), (1,)), ((), ()))
    ps = lax.dot_general(h3, oh, dn1, preferred_element_type=f32)
    pc = lax.dot_general(
        jnp.ones((_H, _BLK), f32), oh, dn1, preferred_element_type=f32
    )

    @pl.when(i == 0)
    def _():
        sums[...] = jnp.zeros_like(sums)
        cnts[...] = jnp.zeros_like(cnts)

    sums[...] += ps
    cnts[...] += pc

    @pl.when(i == _NBLK - 1)
    def _():
        dn = (((0,), (0,)), ((), ()))
        pooled = sums[...] / jnp.maximum(cnts[...], 1.0)
        z = jnp.maximum(
            lax.dot_general(m1w[...], pooled, dn, preferred_element_type=f32)
            + m1b[...], 0.0)
        z = jnp.maximum(
            lax.dot_general(m2w[...], z, dn, preferred_element_type=f32)
            + m2b[...], 0.0)
        z = jnp.maximum(
            lax.dot_general(m3w[...], z, dn, preferred_element_type=f32)
            + m3b[...], 0.0)
        bf = (lax.dot_general(z, ow[...], dn, preferred_element_type=f32)
              + ob[...])
        t = jnp.maximum(
            jnp.dot(xl[...], t1w[...], preferred_element_type=f32)
            + t1b[...], 0.0)
        t = jnp.maximum(
            jnp.dot(t, t2w[...], preferred_element_type=f32) + t2b[...], 0.0)
        tf = jnp.dot(t, t3w[...], preferred_element_type=f32) + t3b[...]
        o_ref[...] = bf * tf + bias[...]


def _const_spec(shape):
    return pl.BlockSpec(shape, lambda i: tuple(0 for _ in shape))


_pool_call = pl.pallas_call(
    _pool_body,
    grid=(_NBLK,),
    in_specs=[
        pl.BlockSpec((_H, _BLK), lambda i: (0, i)),
        pl.BlockSpec((_H // 2, _BLK), lambda i: (0, i)),
        pl.BlockSpec((1, _BLK), lambda i: (0, i)),
        _const_spec((_H, 1)),
        _const_spec((_H, 1)),
        pl.BlockSpec((1, _BLK), lambda i: (0, i)),
        _const_spec((_H, _H)),
        _const_spec((_H, 1)),
        _const_spec((_H, 32)),
        _const_spec((32, 1)),
        _const_spec((32, 16)),
        _const_spec((16, 1)),
        _const_spec((16, 2)),
        _const_spec((1, 2)),
        _const_spec((2, 128)),
        _const_spec((1, 128)),
        _const_spec((128, 256)),
        _const_spec((1, 256)),
        _const_spec((256, 2)),
        _const_spec((1, 2)),
        _const_spec((_G, 2)),
        _const_spec((1, 2)),
    ],
    out_specs=pl.BlockSpec((_G, 2), lambda i: (0, 0)),
    out_shape=jax.ShapeDtypeStruct((_G, 2), jnp.float32),
    scratch_shapes=[
        pltpu.VMEM((_H, _G), jnp.float32),
        pltpu.VMEM((_H, _G), jnp.float32),
    ],
)


def kernel(x, edge_index, batch, x_loc, params):
    p = params
    pk = edge_index[0] + edge_index[1] * _PACK
    xT = jnp.pad(x, ((0, _NPAD - _N), (0, 0))).T
    bp = jnp.pad(batch, (0, _NPAD - _N), constant_values=_G).reshape(1, _NPAD)

    def fold(g, be, b):
        ga = (g * _BN_R).reshape(_H, 1)
        cb = (b * g * _BN_R + be).reshape(_H, 1)
        return ga, cb

    ga1, cb1 = fold(p["g1"], p["be1"], p["b1"])
    ga2, cb2 = fold(p["g2"], p["be2"], p["b2"])
    ga3, cb3 = fold(p["g3"], p["be3"], p["b3"])

    hist = _deg_call(pk)
    s1, dinv = _prologue_call(xT, hist, p["W1"])
    a1 = _edge_call(pk, s1)
    s2 = _mid_call(a1, s1, dinv, ga1, cb1, p["W2"])
    a2 = _edge_call(pk, s2)
    s3 = _mid_call(a2, s2, dinv, ga2, cb2, p["W3"])
    a3 = _edge_call(pk, s3)
    out = _pool_call(
        a3, s3, dinv, ga3, cb3, bp,
        p["m1W"], p["m1b"].reshape(-1, 1),
        p["m2W"], p["m2b"].reshape(-1, 1),
        p["m3W"], p["m3b"].reshape(-1, 1),
        p["oW"], p["ob"].reshape(1, -1),
        p["t1W"], p["t1b"].reshape(1, -1),
        p["t2W"], p["t2b"].reshape(1, -1),
        p["t3W"], p["t3b"].reshape(1, -1),
        x_loc, p["bias"].reshape(1, -1),
    )
    return out
